# Initial kernel scaffold; baseline (speedup 1.0000x reference)
import functools

import jax
import jax.numpy as jnp
from jax import lax
from jax.experimental import pallas as pl
from jax.experimental.pallas import tpu as pltpu
from jax.experimental.pallas import tpu_sc as plsc

N_NODES = 50000
N_EDGES = 1600000
D_NODE = 256
D_EDGE = 16
D_GLOBAL = 16

NC = 2
NS = 16
NW = NC * NS

IDX_ROW = 128
N_ROWS = N_EDGES // IDX_ROW
ROWS_BASE = N_ROWS // NW
ROWS_EXTRA = N_ROWS - ROWS_BASE * NW

GROUP_ROWS = 15
GROUP_EDGES = GROUP_ROWS * IDX_ROW
N_GROUPS = ROWS_BASE // GROUP_ROWS

ACC_ROWS = 50048
SLICE = ACC_ROWS // NS


def _sc_body(edges_hbm, dst_hbm, zeros_hbm, ones_hbm, psum_hbm, pcnt_hbm,
             acc_sum, acc_cnt, ebuf, ibuf, ones_v):
    c = lax.axis_index("c")
    s = lax.axis_index("s")
    w = c * NS + s

    pltpu.sync_copy(zeros_hbm, acc_sum.at[pl.ds(s * SLICE, SLICE)])
    pltpu.sync_copy(zeros_hbm, acc_cnt.at[pl.ds(s * SLICE, SLICE)])
    pltpu.sync_copy(ones_hbm, ones_v)
    plsc.subcore_barrier()

    row_start = ROWS_BASE * w + jnp.minimum(w, ROWS_EXTRA)

    def group(g, carry):
        r0 = row_start + g * GROUP_ROWS
        pltpu.sync_copy(edges_hbm.at[pl.ds(r0 * IDX_ROW, GROUP_EDGES)], ebuf)
        pltpu.sync_copy(dst_hbm.at[pl.ds(r0, GROUP_ROWS)], ibuf)
        for j in range(GROUP_ROWS):
            pltpu.sync_copy(ebuf.at[pl.ds(j * IDX_ROW, IDX_ROW)],
                            acc_sum.at[ibuf.at[j]], add=True)
            pltpu.sync_copy(ones_v, acc_cnt.at[ibuf.at[j]], add=True)
        return carry

    lax.fori_loop(0, N_GROUPS, group, 0)

    @pl.when(w < ROWS_EXTRA)
    def _():
        r0 = row_start + ROWS_BASE
        pltpu.sync_copy(edges_hbm.at[pl.ds(r0 * IDX_ROW, IDX_ROW)],
                        ebuf.at[pl.ds(0, IDX_ROW)])
        pltpu.sync_copy(dst_hbm.at[pl.ds(r0, 1)], ibuf.at[pl.ds(0, 1)])
        pltpu.sync_copy(ebuf.at[pl.ds(0, IDX_ROW)],
                        acc_sum.at[ibuf.at[0]], add=True)
        pltpu.sync_copy(ones_v, acc_cnt.at[ibuf.at[0]], add=True)

    plsc.subcore_barrier()

    pltpu.sync_copy(acc_sum.at[pl.ds(s * SLICE, SLICE)],
                    psum_hbm.at[c, pl.ds(s * SLICE, SLICE)])
    pltpu.sync_copy(acc_cnt.at[pl.ds(s * SLICE, SLICE)],
                    pcnt_hbm.at[c, pl.ds(s * SLICE, SLICE)])


_sc_aggregate = pl.kernel(
    _sc_body,
    out_type=(jax.ShapeDtypeStruct((NC, ACC_ROWS, D_EDGE), jnp.float32),
              jax.ShapeDtypeStruct((NC, ACC_ROWS, D_EDGE), jnp.float32)),
    mesh=plsc.VectorSubcoreMesh(core_axis_name="c", subcore_axis_name="s",
                                num_cores=NC, num_subcores=NS),
    scratch_types=[
        pltpu.VMEM_SHARED((ACC_ROWS, D_EDGE), jnp.float32),
        pltpu.VMEM_SHARED((ACC_ROWS, D_EDGE), jnp.float32),
        pltpu.VMEM((GROUP_EDGES, D_EDGE), jnp.float32),
        pltpu.VMEM((GROUP_ROWS, IDX_ROW), jnp.int32),
        pltpu.VMEM((IDX_ROW, D_EDGE), jnp.float32),
    ],
)

BN = 400


def _tc_body(psum_ref, pcnt_ref, nodes_ref, g_ref, out_ref):
    sums = psum_ref[0] + psum_ref[1]
    cnts = pcnt_ref[0] + pcnt_ref[1]
    mean = sums / jnp.maximum(cnts, 1.0)
    g = jnp.broadcast_to(g_ref[...], (BN, D_GLOBAL))
    out_ref[...] = jnp.concatenate([mean, nodes_ref[...], g], axis=1)


def kernel(nodes_data, edges_data, global_data, edge_index):
    dst = edge_index[1].astype(jnp.int32).reshape(N_ROWS, IDX_ROW)
    zeros_blk = jnp.zeros((SLICE, D_EDGE), jnp.float32)
    ones_blk = jnp.ones((IDX_ROW, D_EDGE), jnp.float32)

    psum, pcnt = _sc_aggregate(edges_data, dst, zeros_blk, ones_blk)

    out = pl.pallas_call(
        _tc_body,
        grid=(N_NODES // BN,),
        in_specs=[
            pl.BlockSpec((NC, BN, D_EDGE), lambda i: (0, i, 0)),
            pl.BlockSpec((NC, BN, D_EDGE), lambda i: (0, i, 0)),
            pl.BlockSpec((BN, D_NODE), lambda i: (i, 0)),
            pl.BlockSpec((1, D_GLOBAL), lambda i: (0, 0)),
        ],
        out_specs=pl.BlockSpec((BN, D_NODE + 2 * D_EDGE), lambda i: (i, 0)),
        out_shape=jax.ShapeDtypeStruct((N_NODES, D_NODE + 2 * D_EDGE),
                                       jnp.float32),
    )(psum, pcnt, nodes_data, global_data.reshape(1, D_GLOBAL))
    return out


# speedup vs baseline: 6.5716x; 6.5716x over previous
import functools

import jax
import jax.numpy as jnp
from jax import lax
from jax.experimental import pallas as pl
from jax.experimental.pallas import tpu as pltpu
from jax.experimental.pallas import tpu_sc as plsc

N_NODES = 50000
N_EDGES = 1600000
D_NODE = 256
D_EDGE = 16
D_GLOBAL = 16

NC = 2
NS = 16
NW = NC * NS

IDX_ROW = 128
N_ROWS = N_EDGES // IDX_ROW
ROWS_BASE = N_ROWS // NW
ROWS_EXTRA = N_ROWS - ROWS_BASE * NW

GROUP_ROWS = 15
GROUP_EDGES = GROUP_ROWS * IDX_ROW
N_GROUPS = ROWS_BASE // GROUP_ROWS

ACC_ROWS = 50048
SLICE = ACC_ROWS // NS


def _sc_body(edges_hbm, dst_hbm, zeros_hbm, zeros16_hbm, ones_hbm,
             psum_hbm, pcnt_hbm, acc_sum, acc_cnt, ebuf, ibuf, ones_v):
    c = lax.axis_index("c")
    s = lax.axis_index("s")
    w = c * NS + s

    pltpu.sync_copy(zeros_hbm, acc_sum.at[pl.ds(s * SLICE, SLICE)])
    pltpu.sync_copy(zeros16_hbm, acc_cnt.at[pl.ds(s * SLICE, SLICE)])
    pltpu.sync_copy(ones_hbm, ones_v)
    plsc.subcore_barrier()

    row_start = ROWS_BASE * w + jnp.minimum(w, ROWS_EXTRA)

    def group(g, carry):
        r0 = row_start + g * GROUP_ROWS
        pltpu.sync_copy(edges_hbm.at[pl.ds(r0 * IDX_ROW, GROUP_EDGES)], ebuf)
        pltpu.sync_copy(dst_hbm.at[pl.ds(r0, GROUP_ROWS)], ibuf)
        for j in range(GROUP_ROWS):
            pltpu.sync_copy(ebuf.at[pl.ds(j * IDX_ROW, IDX_ROW)],
                            acc_sum.at[ibuf.at[j, 0]], add=True)
            pltpu.sync_copy(ones_v, acc_cnt.at[ibuf.at[j, 0]], add=True)
        return carry

    lax.fori_loop(0, N_GROUPS, group, 0)

    @pl.when(w < ROWS_EXTRA)
    def _():
        r0 = row_start + ROWS_BASE
        pltpu.sync_copy(edges_hbm.at[pl.ds(r0 * IDX_ROW, IDX_ROW)],
                        ebuf.at[pl.ds(0, IDX_ROW)])
        pltpu.sync_copy(dst_hbm.at[pl.ds(r0, 1)], ibuf.at[pl.ds(0, 1)])
        pltpu.sync_copy(ebuf.at[pl.ds(0, IDX_ROW)],
                        acc_sum.at[ibuf.at[0, 0]], add=True)
        pltpu.sync_copy(ones_v, acc_cnt.at[ibuf.at[0, 0]], add=True)

    plsc.subcore_barrier()

    pltpu.sync_copy(acc_sum.at[pl.ds(s * SLICE, SLICE)],
                    psum_hbm.at[c, pl.ds(s * SLICE, SLICE)])
    pltpu.sync_copy(acc_cnt.at[pl.ds(s * SLICE, SLICE)],
                    pcnt_hbm.at[c, pl.ds(s * SLICE, SLICE)])


_sc_aggregate = pl.kernel(
    _sc_body,
    out_type=(jax.ShapeDtypeStruct((NC, ACC_ROWS, D_EDGE), jnp.float32),
              jax.ShapeDtypeStruct((NC, ACC_ROWS, D_EDGE), jnp.int16)),
    mesh=plsc.VectorSubcoreMesh(core_axis_name="c", subcore_axis_name="s",
                                num_cores=NC, num_subcores=NS),
    scratch_types=[
        pltpu.VMEM_SHARED((ACC_ROWS, D_EDGE), jnp.float32),
        pltpu.VMEM_SHARED((ACC_ROWS, D_EDGE), jnp.int16),
        pltpu.VMEM((GROUP_EDGES, D_EDGE), jnp.float32),
        pltpu.VMEM((GROUP_ROWS, 1, IDX_ROW), jnp.int32),
        pltpu.VMEM((IDX_ROW, D_EDGE), jnp.int16),
    ],
    compiler_params=pltpu.CompilerParams(use_tc_tiling_on_sc=False),
)

BN = 400


def _tc_body(psum_ref, pcnt_ref, nodes_ref, g_ref, out_ref):
    sums = psum_ref[0] + psum_ref[1]
    cnts = (pcnt_ref[0].astype(jnp.float32) +
            pcnt_ref[1].astype(jnp.float32))
    mean = sums / jnp.maximum(cnts, 1.0)
    g = jnp.broadcast_to(g_ref[...], (BN, D_GLOBAL))
    out_ref[...] = jnp.concatenate([mean, nodes_ref[...], g], axis=1)


def kernel(nodes_data, edges_data, global_data, edge_index):
    dst = edge_index[1].astype(jnp.int32).reshape(N_ROWS, 1, IDX_ROW)
    zeros_blk = jnp.zeros((SLICE, D_EDGE), jnp.float32)
    zeros_blk_i16 = jnp.zeros((SLICE, D_EDGE), jnp.int16)
    ones_blk = jnp.ones((IDX_ROW, D_EDGE), jnp.int16)

    psum, pcnt = _sc_aggregate(edges_data, dst, zeros_blk, zeros_blk_i16,
                               ones_blk)

    out = pl.pallas_call(
        _tc_body,
        grid=(N_NODES // BN,),
        in_specs=[
            pl.BlockSpec((NC, BN, D_EDGE), lambda i: (0, i, 0)),
            pl.BlockSpec((NC, BN, D_EDGE), lambda i: (0, i, 0)),
            pl.BlockSpec((BN, D_NODE), lambda i: (i, 0)),
            pl.BlockSpec((1, D_GLOBAL), lambda i: (0, 0)),
        ],
        out_specs=pl.BlockSpec((BN, D_NODE + 2 * D_EDGE), lambda i: (i, 0)),
        out_shape=jax.ShapeDtypeStruct((N_NODES, D_NODE + 2 * D_EDGE),
                                       jnp.float32),
    )(psum, pcnt, nodes_data, global_data.reshape(1, D_GLOBAL))
    return out
